# baseline (speedup 1.0000x reference)
import jax
import jax.numpy as jnp
from jax.experimental import pallas as pl
from jax.experimental.pallas import tpu as pltpu

_N = 16384
_H = 1024
_W = 1024


def _tail_kernel(idx_ref, colg_ref, cx_ref, cy_ref, col_ref):
    idx = idx_ref[...]
    cx = jax.lax.rem(idx, _W)
    cy = jax.lax.div(idx, _H)
    fx = cx.astype(jnp.float32) / _H
    fy = cy.astype(jnp.float32) / _H
    cx_ref[...] = jnp.clip(fx, 0.0, 1.0)
    cy_ref[...] = jnp.clip(fy, 0.0, 1.0)
    col_ref[...] = jnp.clip(colg_ref[...], 0.0, 1.0)


def kernel(target, canvas):
    error_map = jnp.mean(jnp.abs(target - canvas), axis=0)
    flat = error_map.reshape(-1) / 1.0
    logp = jax.nn.log_softmax(flat, axis=0)
    u = jax.random.uniform(jax.random.key(42), logp.shape,
                           minval=1e-20, maxval=1.0)
    gumbel = -jnp.log(-jnp.log(u))
    _, indices = jax.lax.top_k(logp + gumbel, _N)
    indices = jax.lax.stop_gradient(indices)

    colg = jnp.take(target.reshape(3, -1), indices, axis=1)

    idx2 = indices.reshape(128, 128)
    cx, cy, col = pl.pallas_call(
        _tail_kernel,
        out_shape=(
            jax.ShapeDtypeStruct((128, 128), jnp.float32),
            jax.ShapeDtypeStruct((128, 128), jnp.float32),
            jax.ShapeDtypeStruct((3, _N), jnp.float32),
        ),
    )(idx2, colg)

    center_x = cx.reshape(_N, 1)
    center_y = cy.reshape(_N, 1)
    color = col.reshape(_N, 3, 1, 1)
    return center_x, center_y, color


# speedup vs baseline: 7.2110x; 7.2110x over previous
import functools

import numpy as np
import jax
import jax.numpy as jnp
from jax import lax
from jax.experimental import pallas as pl
from jax.experimental.pallas import tpu as pltpu
from jax.experimental.pallas import tpu_sc as plsc

_N = 16384
_NPIX = 1 << 20
_H = 1024
_W = 1024
_NT = 16


def _np_threefry_uniform(n):
    k1, k2 = np.uint32(0), np.uint32(42)
    x0 = np.zeros(n, np.uint32)
    x1 = np.arange(n, dtype=np.uint32)
    ks = [k1, k2, np.uint32(k1 ^ k2 ^ np.uint32(0x1BD11BDA))]
    rots = [[13, 15, 26, 6], [17, 29, 16, 24]]

    def rounds(x0, x1, rs):
        for r in rs:
            x0 = (x0 + x1).astype(np.uint32)
            x1 = (x1 << np.uint32(r)) | (x1 >> np.uint32(32 - r))
            x1 = x0 ^ x1
        return x0, x1

    x0 = (x0 + ks[0]).astype(np.uint32)
    x1 = (x1 + ks[1]).astype(np.uint32)
    add1 = [ks[1], ks[2], ks[0], ks[1], ks[2]]
    add2 = [ks[2], ks[0], ks[1], ks[2], ks[0]]
    for i in range(5):
        x0, x1 = rounds(x0, x1, rots[i % 2])
        x0 = (x0 + add1[i]).astype(np.uint32)
        x1 = (x1 + add2[i] + np.uint32(i + 1)).astype(np.uint32)
    bits = x0 ^ x1
    fb = (bits >> np.uint32(9)) | np.uint32(0x3F800000)
    f = fb.view(np.float32) - np.float32(1.0)
    mn, mx = np.float32(1e-20), np.float32(1.0)
    return np.maximum(mn, f * (mx - mn) + mn)


def _precompute_candidates():
    u = _np_threefry_uniform(_NPIX).astype(np.float64)
    g = -np.log(-np.log(u))
    kth = np.partition(g, _NPIX - _N)[_NPIX - _N]
    cand = np.nonzero(g >= kth - 1.0 - 0.03)[0].astype(np.int32)
    nc = int(cand.size)
    npad = -(-nc // 16384) * 16384
    return np.concatenate([cand, np.zeros(npad - nc, np.int32)]), nc, npad


_CAND, _NC, _NPAD = _precompute_candidates()
_C = _NPAD // _NT
_RC = _C // 128
_NV = _C // 16
_CAND2 = _CAND.reshape(_NPAD // 128, 128)


def _sc_body(s_ref, cand2_ref, cand1_ref, tgt_ref, cx_ref, cy_ref, col_ref,
             kA, pA, kB, pB, grid, candv, sflat, kbuf, pbuf, hist, offs,
             gridv, st32, ibuf, pixbuf, fxbuf, fybuf, colbuf, sem):
    w = lax.axis_index("s")
    core = lax.axis_index("c")
    base = w * _C
    zeros16 = jnp.zeros((16,), jnp.int32)
    ones16 = jnp.ones((16,), jnp.int32)
    fzeros16 = jnp.zeros((16,), jnp.float32)
    fones16 = jnp.ones((16,), jnp.float32)

    @pl.when(core == 0)
    def _run():
        pltpu.sync_copy(cand2_ref.at[pl.ds(w * _RC, _RC)], candv)
        cps = [pltpu.async_copy(s_ref.at[candv.at[r]],
                                sflat.at[pl.ds(r * 128, 128)], sem)
               for r in range(_RC)]
        for cp in cps:
            cp.wait()

        def keyfn(i, carry):
            b = sflat[pl.ds(i * 16, 16)]
            keyv = jnp.where(b >= 0, b ^ jnp.int32(0x7FFFFFFF), b)
            j = base + i * 16 + lax.iota(jnp.int32, 16)
            keyv = jnp.where(j < _NC, keyv, jnp.int32(-1))
            kbuf[pl.ds(i * 16, 16)] = keyv
            pbuf[pl.ds(i * 16, 16)] = j
            return carry

        lax.fori_loop(0, _NV, keyfn, 0)
        pltpu.sync_copy(kbuf, kA.at[pl.ds(base, _C)])
        pltpu.sync_copy(pbuf, pA.at[pl.ds(base, _C)])
        plsc.subcore_barrier()

        def lane_rank(dv):
            st32[pl.ds(16, 16)] = dv
            rank = zeros16
            for kk in range(1, 16):
                sv = st32[pl.ds(16 - kk, 16)]
                rank = rank + jnp.where(sv == dv, ones16, zeros16)
            return rank

        def radix_pass(sh, srcK, srcP, dstK, dstP):
            pltpu.sync_copy(srcK.at[pl.ds(base, _C)], kbuf)
            pltpu.sync_copy(srcP.at[pl.ds(base, _C)], pbuf)
            for bq in range(16):
                hist[pl.ds(bq * 16, 16)] = zeros16
            st32[pl.ds(0, 16)] = jnp.full((16,), 999, jnp.int32)

            def hf(i, carry):
                kv = kbuf[pl.ds(i * 16, 16)]
                dv = lax.shift_right_logical(kv, sh) & 255
                rank = lane_rank(dv)
                hv = plsc.load_gather(hist, [dv])
                plsc.store_scatter(hist, [dv], hv + rank + ones16)
                return carry

            lax.fori_loop(0, _NV, hf, 0)
            pltpu.sync_copy(hist, grid.at[pl.ds(w * 256, 256)])
            plsc.subcore_barrier()
            pltpu.sync_copy(grid, gridv)

            for bq in range(16):
                def rowf(t, carry):
                    tot, bef = carry
                    row = gridv[pl.ds(t * 256 + bq * 16, 16)]
                    tot = tot + row
                    bef = bef + jnp.where(t < w, row, zeros16)
                    return (tot, bef)

                tot, bef = lax.fori_loop(0, _NT, rowf, (zeros16, zeros16))
                hist[pl.ds(bq * 16, 16)] = bef
                offs[pl.ds(bq * 16, 16)] = tot

            carry = jnp.int32(0)
            for bq in range(16):
                tv = offs[pl.ds(bq * 16, 16)]
                inc = plsc.cumsum(tv)
                excl = inc - tv + carry
                bef = hist[pl.ds(bq * 16, 16)]
                offs[pl.ds(bq * 16, 16)] = excl + bef
                carry = carry + jnp.sum(tv)

            def scf(i, carry):
                kv = kbuf[pl.ds(i * 16, 16)]
                dv = lax.shift_right_logical(kv, sh) & 255
                rank = lane_rank(dv)
                off16 = plsc.load_gather(offs, [dv])
                pos = off16 + rank
                plsc.store_scatter(offs, [dv], pos + ones16)
                pltpu.async_copy(kbuf.at[pl.ds(i * 16, 16)], dstK.at[pos], sem)
                pltpu.async_copy(pbuf.at[pl.ds(i * 16, 16)], dstP.at[pos], sem)
                return carry

            lax.fori_loop(0, _NV, scf, 0)
            pltpu.make_async_copy(cand1_ref.at[pl.ds(0, _C)], kbuf, sem).wait()
            pltpu.make_async_copy(cand1_ref.at[pl.ds(0, _C)], pbuf, sem).wait()
            plsc.subcore_barrier()

        radix_pass(0, kA, pA, kB, pB)
        radix_pass(8, kB, pB, kA, pA)
        radix_pass(16, kA, pA, kB, pB)
        radix_pass(24, kB, pB, kA, pA)

        pltpu.sync_copy(pA.at[pl.ds(w * 1024, 1024)], ibuf)
        cps = [pltpu.async_copy(cand1_ref.at[ibuf.at[pl.ds(r * 128, 128)]],
                                pixbuf.at[pl.ds(r * 128, 128)], sem)
               for r in range(8)]
        for cp in cps:
            cp.wait()

        def cenf(i, carry):
            p = pixbuf[pl.ds(i * 16, 16)]
            cx = p & jnp.int32(_W - 1)
            cy = lax.shift_right_logical(p, 10)
            fx = cx.astype(jnp.float32) * jnp.float32(1.0 / _H)
            fy = cy.astype(jnp.float32) * jnp.float32(1.0 / _H)
            fxbuf[pl.ds(i * 16, 16)] = jnp.clip(fx, 0.0, 1.0)
            fybuf[pl.ds(i * 16, 16)] = jnp.clip(fy, 0.0, 1.0)
            return carry

        lax.fori_loop(0, 64, cenf, 0)
        pltpu.sync_copy(fxbuf, cx_ref.at[w])
        pltpu.sync_copy(fybuf, cy_ref.at[w])

        for c in range(3):
            def colif(i, carry):
                ibuf[pl.ds(i * 16, 16)] = (pixbuf[pl.ds(i * 16, 16)]
                                           + jnp.int32(c * _NPIX))
                return carry

            lax.fori_loop(0, 64, colif, 0)
            cps = [pltpu.async_copy(tgt_ref.at[ibuf.at[pl.ds(r * 128, 128)]],
                                    colbuf.at[pl.ds(r * 128, 128)], sem)
                   for r in range(8)]
            for cp in cps:
                cp.wait()
            pltpu.sync_copy(colbuf, col_ref.at[c, w])


_sc_kernel = pl.kernel(
    _sc_body,
    out_type=(
        jax.ShapeDtypeStruct((_NT, 1024), jnp.float32),
        jax.ShapeDtypeStruct((_NT, 1024), jnp.float32),
        jax.ShapeDtypeStruct((3, _NT, 1024), jnp.float32),
    ),
    mesh=plsc.VectorSubcoreMesh(core_axis_name="c", subcore_axis_name="s"),
    compiler_params=pltpu.CompilerParams(needs_layout_passes=False),
    scratch_types=[
        pltpu.VMEM_SHARED((_NPAD,), jnp.int32),
        pltpu.VMEM_SHARED((_NPAD,), jnp.int32),
        pltpu.VMEM_SHARED((_NPAD,), jnp.int32),
        pltpu.VMEM_SHARED((_NPAD,), jnp.int32),
        pltpu.VMEM_SHARED((_NT * 256,), jnp.int32),
        pltpu.VMEM((_RC, 128), jnp.int32),
        pltpu.VMEM((_C,), jnp.int32),
        pltpu.VMEM((_C,), jnp.int32),
        pltpu.VMEM((_C,), jnp.int32),
        pltpu.VMEM((256,), jnp.int32),
        pltpu.VMEM((256,), jnp.int32),
        pltpu.VMEM((_NT * 256,), jnp.int32),
        pltpu.VMEM((32,), jnp.int32),
        pltpu.VMEM((1024,), jnp.int32),
        pltpu.VMEM((1024,), jnp.int32),
        pltpu.VMEM((1024,), jnp.float32),
        pltpu.VMEM((1024,), jnp.float32),
        pltpu.VMEM((1024,), jnp.float32),
        pltpu.SemaphoreType.DMA,
    ],
)


def kernel(target, canvas):
    error_map = jnp.mean(jnp.abs(target - canvas), axis=0)
    flat = error_map.reshape(-1) / 1.0
    logp = jax.nn.log_softmax(flat, axis=0)
    u = jax.random.uniform(jax.random.key(42), logp.shape,
                           minval=1e-20, maxval=1.0)
    gumbel = -jnp.log(-jnp.log(u))
    s = logp + gumbel

    cx, cy, col = _sc_kernel(
        lax.bitcast_convert_type(s, jnp.int32),
        jnp.asarray(_CAND2),
        jnp.asarray(_CAND),
        target.reshape(-1),
    )
    center_x = cx.reshape(_N, 1)
    center_y = cy.reshape(_N, 1)
    color = col.reshape(_N, 3, 1, 1)
    return center_x, center_y, color
